# baseline (speedup 1.0000x reference)
import jax
import jax.numpy as jnp
from jax import lax
from jax.experimental import pallas as pl
from jax.experimental.pallas import tpu as pltpu
from jax.experimental.pallas import tpu_sc as plsc

N = 10000
NP = 10240
E = 320000
D = 128
DH = 256
NC, NS, L = 2, 16, 16
NW = NC * NS
CH = 128
KROW = 2
N_CHUNKS = 2560
CPT = N_CHUNKS // NW
E_PAD = N_CHUNKS * CH
RPT = NP // NS
SCPT = 26
S_PAD = NW * SCPT * CH
BLK = 1024


def _mesh():
    return plsc.VectorSubcoreMesh(
        core_axis_name="c", subcore_axis_name="s", num_cores=NC, num_subcores=NS
    )


def _make_segsum(with_deg: bool):
    out_type = [jax.ShapeDtypeStruct((NC, NP, D), jnp.bfloat16)]
    nsc = CPT // KROW
    scratch = [
        pltpu.VMEM((nsc, KROW * CH), jnp.int32),
        pltpu.VMEM((nsc, KROW * CH), jnp.int32),
        [pltpu.VMEM((KROW * CH, D), jnp.bfloat16)] * 2,
        pltpu.SemaphoreType.DMA,
        pltpu.SemaphoreType.DMA,
        pltpu.VMEM_SHARED((NP, D), jnp.bfloat16),
    ]
    if with_deg:
        out_type.append(jax.ShapeDtypeStruct((NC, NP, L), jnp.float32))
        scratch += [
            pltpu.VMEM((KROW * CH, L), jnp.float32),
            pltpu.VMEM_SHARED((NP, L), jnp.float32),
        ]

    def body(vals, srci2, dsti2, z128, z16, ones_h, *rest):
        if with_deg:
            parts_o, deg_o, idxs, idxd, bufs, semA, semB, acc, ones_v, dacc = rest
        else:
            parts_o, idxs, idxd, bufs, semA, semB, acc = rest
        rowsA, rowsB = bufs
        c = lax.axis_index("c")
        s = lax.axis_index("s")
        w = c * NS + s
        sl = pl.ds(s * RPT, RPT)
        pltpu.sync_copy(z128.at[sl], acc.at[sl])
        if with_deg:
            pltpu.sync_copy(z16.at[sl], dacc.at[sl])
            pltpu.sync_copy(ones_h, ones_v)
        nsc_l = CPT // KROW
        pltpu.sync_copy(srci2.at[pl.ds(w * nsc_l, nsc_l)], idxs)
        pltpu.sync_copy(dsti2.at[pl.ds(w * nsc_l, nsc_l)], idxd)
        plsc.subcore_barrier()

        def start_g(k, rbuf, sem):
            pltpu.async_copy(vals.at[idxs.at[k]], rbuf, sem)

        def wait_g(k, rbuf, sem):
            pltpu.make_async_copy(vals.at[idxs.at[k]], rbuf, sem).wait()

        def scat(rbuf, k):
            pltpu.sync_copy(rbuf, acc.at[idxd.at[k]], add=True)
            if with_deg:
                pltpu.sync_copy(ones_v, dacc.at[idxd.at[k]], add=True)

        start_g(0, rowsA, semA)

        def it(i, cc):
            start_g(2 * i + 1, rowsB, semB)
            wait_g(2 * i, rowsA, semA)
            scat(rowsA, 2 * i)

            @pl.when(i < nsc_l // 2 - 1)
            def _():
                start_g(2 * i + 2, rowsA, semA)

            wait_g(2 * i + 1, rowsB, semB)
            scat(rowsB, 2 * i + 1)
            return cc

        lax.fori_loop(0, nsc_l // 2, it, 0)
        plsc.subcore_barrier()
        pltpu.sync_copy(acc.at[sl], parts_o.at[c, sl])
        if with_deg:
            pltpu.sync_copy(dacc.at[sl], deg_o.at[c, sl])

    return pl.kernel(body, out_type=out_type, mesh=_mesh(), scratch_types=scratch,
                     compiler_params=pltpu.CompilerParams(use_tc_tiling_on_sc=False))


def _score_kernel():
    out_type = [jax.ShapeDtypeStruct((S_PAD // CH, CH), jnp.float32)]
    scratch = [
        pltpu.VMEM((SCPT, CH), jnp.int32),
        pltpu.VMEM((SCPT, CH), jnp.int32),
        [pltpu.VMEM((CH, D), jnp.bfloat16)] * 2,
        [pltpu.VMEM((CH, D), jnp.bfloat16)] * 2,
        pltpu.VMEM((SCPT, CH), jnp.float32),
        pltpu.SemaphoreType.DMA,
        pltpu.SemaphoreType.DMA,
    ]

    def body(h2, ui2, vi2, sc_o, idxu, idxv, ub, vb, sres, semu, semv):
        c = lax.axis_index("c")
        s = lax.axis_index("s")
        w = c * NS + s
        pltpu.sync_copy(ui2.at[pl.ds(w * SCPT, SCPT)], idxu)
        pltpu.sync_copy(vi2.at[pl.ds(w * SCPT, SCPT)], idxv)

        lane = lax.iota(jnp.int32, L)
        sems = [semu, semv]

        def start_g(k, j):
            pltpu.async_copy(h2.at[idxu.at[k]], ub[j], sems[j])
            pltpu.async_copy(h2.at[idxv.at[k]], vb[j], sems[j])

        def wait_g(k, j):
            pltpu.make_async_copy(h2.at[idxu.at[k]], ub[j], sems[j]).wait()
            pltpu.make_async_copy(h2.at[idxv.at[k]], vb[j], sems[j]).wait()

        def compute(k, j):
            u, v = ub[j], vb[j]

            def grp(g, gc):
                vec = jnp.zeros((L,), jnp.float32)
                for eo in range(L):
                    e = g * L + eo
                    prods = []
                    for q in range(D // 32):
                        uu = u[e, pl.ds(q * 32, 32)]
                        vv = v[e, pl.ds(q * 32, 32)]
                        ua, ub2 = plsc.unpack(uu, format=plsc.PackFormat.INTERLEAVED)
                        va, vb2 = plsc.unpack(vv, format=plsc.PackFormat.INTERLEAVED)
                        prods.append(ua * va)
                        prods.append(ub2 * vb2)
                    a = ((prods[0] + prods[1]) + (prods[2] + prods[3])) + \
                        ((prods[4] + prods[5]) + (prods[6] + prods[7]))
                    vec = jnp.where(lane == eo, jnp.sum(a), vec)
                sres[k, pl.ds(g * L, L)] = vec
                return gc

            lax.fori_loop(0, CH // L, grp, 0)

        start_g(0, 0)

        def it(i, cc):
            start_g(2 * i + 1, 1)
            wait_g(2 * i, 0)
            compute(2 * i, 0)

            @pl.when(i < SCPT // 2 - 1)
            def _():
                start_g(2 * i + 2, 0)

            wait_g(2 * i + 1, 1)
            compute(2 * i + 1, 1)
            return cc

        lax.fori_loop(0, SCPT // 2, it, 0)
        pltpu.sync_copy(sres, sc_o.at[pl.ds(w * SCPT, SCPT)])

    return pl.kernel(body, out_type=out_type, mesh=_mesh(), scratch_types=scratch,
                     compiler_params=pltpu.CompilerParams(
                         use_tc_tiling_on_sc=False, needs_layout_passes=False))


def _t1_body(parts, deg, x, ws0, wn0, ws1, wn1, s1_o, p1_o):
    dg = deg[...]
    d = jnp.maximum(dg[0, :, 0:1] + dg[1, :, 0:1], 1.0)
    rd = 1.0 / d
    p = parts[...].astype(jnp.float32)
    agg = (p[0] + p[1]) * rd
    h1 = jnp.dot(x[...], ws0[...], preferred_element_type=jnp.float32)
    h1 = h1 + jnp.dot(agg, wn0[...], preferred_element_type=jnp.float32)
    h1 = jnp.maximum(h1, 0.0)
    s1_o[...] = jnp.dot(h1, ws1[...], preferred_element_type=jnp.float32)
    p1_o[...] = jnp.dot(h1, wn1[...],
                        preferred_element_type=jnp.float32).astype(jnp.bfloat16)


def _t2_body(s1, parts, deg, h2_o):
    dg = deg[...]
    d = jnp.maximum(dg[0, :, 0:1] + dg[1, :, 0:1], 1.0)
    rd = 1.0 / d
    p = parts[...].astype(jnp.float32)
    h2_o[...] = (s1[...] + (p[0] + p[1]) * rd).astype(jnp.bfloat16)


_t1_call = pl.pallas_call(
    _t1_body,
    grid=(NP // BLK,),
    in_specs=[
        pl.BlockSpec((NC, BLK, D), lambda i: (0, i, 0)),
        pl.BlockSpec((NC, BLK, L), lambda i: (0, i, 0)),
        pl.BlockSpec((BLK, D), lambda i: (i, 0)),
        pl.BlockSpec((D, DH), lambda i: (0, 0)),
        pl.BlockSpec((D, DH), lambda i: (0, 0)),
        pl.BlockSpec((DH, D), lambda i: (0, 0)),
        pl.BlockSpec((DH, D), lambda i: (0, 0)),
    ],
    out_specs=[
        pl.BlockSpec((BLK, D), lambda i: (i, 0)),
        pl.BlockSpec((BLK, D), lambda i: (i, 0)),
    ],
    out_shape=[
        jax.ShapeDtypeStruct((NP, D), jnp.float32),
        jax.ShapeDtypeStruct((NP, D), jnp.bfloat16),
    ],
)

_t2_call = pl.pallas_call(
    _t2_body,
    grid=(NP // BLK,),
    in_specs=[
        pl.BlockSpec((BLK, D), lambda i: (i, 0)),
        pl.BlockSpec((NC, BLK, D), lambda i: (0, i, 0)),
        pl.BlockSpec((NC, BLK, L), lambda i: (0, i, 0)),
    ],
    out_specs=pl.BlockSpec((BLK, D), lambda i: (i, 0)),
    out_shape=jax.ShapeDtypeStruct((NP, D), jnp.bfloat16),
)


def kernel(x, edge_index, pos_u, pos_v, neg_u, neg_v,
           W_self0, W_neigh0, W_self1, W_neigh1):
    src = edge_index[0].astype(jnp.int32)
    dst = edge_index[1].astype(jnp.int32)
    pad_e = E_PAD - E
    src2 = jnp.concatenate([src, jnp.zeros((pad_e,), jnp.int32)]
                           ).reshape(N_CHUNKS // KROW, KROW * CH)
    pad_rows = N + (jnp.arange(pad_e, dtype=jnp.int32) % (NP - N))
    dst2 = jnp.concatenate([dst, pad_rows]).reshape(N_CHUNKS // KROW, KROW * CH)
    z128 = jnp.zeros((NP, D), jnp.bfloat16)
    z16 = jnp.zeros((NP, L), jnp.float32)
    ones_h = jnp.ones((KROW * CH, L), jnp.float32)
    xp = jnp.concatenate([x, jnp.zeros((NP - N, D), jnp.float32)])
    xb = xp.astype(jnp.bfloat16)

    seg1 = _make_segsum(with_deg=True)
    parts1, deg = seg1(xb, src2, dst2, z128, z16, ones_h)
    s1, p1 = _t1_call(parts1, deg, xp, W_self0, W_neigh0, W_self1, W_neigh1)
    seg2 = _make_segsum(with_deg=False)
    (parts2,) = seg2(p1, src2, dst2, z128, z16, ones_h)
    h2 = _t2_call(s1, parts2, deg)

    n_pos = pos_u.shape[0]
    n_neg = neg_u.shape[0]
    pad_s = S_PAD - n_pos - n_neg
    pad_idx = jnp.arange(pad_s, dtype=jnp.int32) % N
    ui = jnp.concatenate([pos_u.astype(jnp.int32), neg_u.astype(jnp.int32),
                          pad_idx]).reshape(S_PAD // CH, CH)
    vi = jnp.concatenate([pos_v.astype(jnp.int32), neg_v.astype(jnp.int32),
                          pad_idx]).reshape(S_PAD // CH, CH)
    (scores2,) = _score_kernel()(h2, ui, vi)
    scores = scores2.reshape(-1)
    return scores[:n_pos], scores[n_pos:n_pos + n_neg]


# speedup vs baseline: 1.4958x; 1.4958x over previous
import jax
import jax.numpy as jnp
from jax import lax
from jax.experimental import pallas as pl
from jax.experimental.pallas import tpu as pltpu
from jax.experimental.pallas import tpu_sc as plsc

N = 10000
NP = 10240
E = 320000
D = 128
DHALF = D // 2
DH = 256
NC, NS, L = 2, 16, 16
NW = NC * NS
CH = 128
N_CHUNKS = 2560
CPT = N_CHUNKS // NS
E_PAD = N_CHUNKS * CH
RPT = NP // NS
KROW = 4
SCPT = 26
S_PAD = NW * SCPT * CH
BLK = 1024


def _mesh():
    return plsc.VectorSubcoreMesh(
        core_axis_name="c", subcore_axis_name="s", num_cores=NC, num_subcores=NS
    )


def _make_segsum(with_deg: bool):
    out_type = [jax.ShapeDtypeStruct((NC, NP, DHALF), jnp.bfloat16)]
    scratch = [
        pltpu.VMEM((CPT // KROW, KROW * CH), jnp.int32),
        pltpu.VMEM((CPT // KROW, KROW * CH), jnp.int32),
        [pltpu.VMEM((KROW * CH, DHALF), jnp.bfloat16)] * 2,
        pltpu.SemaphoreType.DMA,
        pltpu.SemaphoreType.DMA,
        pltpu.VMEM_SHARED((NP, DHALF), jnp.bfloat16),
    ]
    if with_deg:
        out_type.append(jax.ShapeDtypeStruct((NC, NP, L), jnp.float32))
        scratch += [
            pltpu.VMEM((KROW * CH, L), jnp.float32),
            pltpu.VMEM_SHARED((NP, L), jnp.float32),
        ]

    def body(vals2, srci2, dsti2, z64, z16, ones_h, *rest):
        if with_deg:
            parts_o, deg_o, idxs, idxd, bufs, semA, semB, acc, ones_v, dacc = rest
        else:
            parts_o, idxs, idxd, bufs, semA, semB, acc = rest
        rowsA, rowsB = bufs
        c = lax.axis_index("c")
        s = lax.axis_index("s")
        on_core0 = c == 0
        sl = pl.ds(s * RPT, RPT)
        vals = vals2.at[c]
        pltpu.sync_copy(z64.at[sl], acc.at[sl])
        if with_deg:
            pltpu.sync_copy(z16.at[sl], dacc.at[sl])
            pltpu.sync_copy(ones_h, ones_v)
        nsc = CPT // KROW
        pltpu.sync_copy(srci2.at[pl.ds(s * nsc, nsc)], idxs)
        pltpu.sync_copy(dsti2.at[pl.ds(s * nsc, nsc)], idxd)
        plsc.subcore_barrier()

        def start_g(k, rbuf, sem):
            pltpu.async_copy(vals.at[idxs.at[k]], rbuf, sem)

        def wait_g(k, rbuf, sem):
            pltpu.make_async_copy(vals.at[idxs.at[k]], rbuf, sem).wait()

        def scat(rbuf, k):
            pltpu.sync_copy(rbuf, acc.at[idxd.at[k]], add=True)
            if with_deg:
                @pl.when(on_core0 == (k < CPT // KROW // 2))
                def _():
                    pltpu.sync_copy(ones_v, dacc.at[idxd.at[k]], add=True)

        n_sc = CPT // KROW
        start_g(0, rowsA, semA)

        def it(i, cc):
            start_g(2 * i + 1, rowsB, semB)
            wait_g(2 * i, rowsA, semA)
            scat(rowsA, 2 * i)

            @pl.when(i < n_sc // 2 - 1)
            def _():
                start_g(2 * i + 2, rowsA, semA)

            wait_g(2 * i + 1, rowsB, semB)
            scat(rowsB, 2 * i + 1)
            return cc

        lax.fori_loop(0, n_sc // 2, it, 0)
        plsc.subcore_barrier()
        pltpu.sync_copy(acc.at[sl], parts_o.at[c, sl])
        if with_deg:
            pltpu.sync_copy(dacc.at[sl], deg_o.at[c, sl])

    return pl.kernel(body, out_type=out_type, mesh=_mesh(), scratch_types=scratch,
                     compiler_params=pltpu.CompilerParams(use_tc_tiling_on_sc=False))


def _score_kernel():
    out_type = [jax.ShapeDtypeStruct((S_PAD // CH, CH), jnp.float32)]
    scratch = [
        pltpu.VMEM((SCPT, CH), jnp.int32),
        pltpu.VMEM((SCPT, CH), jnp.int32),
        [pltpu.VMEM((CH, D), jnp.bfloat16)] * 2,
        [pltpu.VMEM((CH, D), jnp.bfloat16)] * 2,
        pltpu.VMEM((SCPT, CH), jnp.float32),
        pltpu.SemaphoreType.DMA,
        pltpu.SemaphoreType.DMA,
    ]

    def body(h2, ui2, vi2, sc_o, idxu, idxv, ub, vb, sres, semu, semv):
        c = lax.axis_index("c")
        s = lax.axis_index("s")
        w = c * NS + s
        pltpu.sync_copy(ui2.at[pl.ds(w * SCPT, SCPT)], idxu)
        pltpu.sync_copy(vi2.at[pl.ds(w * SCPT, SCPT)], idxv)

        lane = lax.iota(jnp.int32, L)
        sems = [semu, semv]

        def start_g(k, j):
            pltpu.async_copy(h2.at[idxu.at[k]], ub[j], sems[j])
            pltpu.async_copy(h2.at[idxv.at[k]], vb[j], sems[j])

        def wait_g(k, j):
            pltpu.make_async_copy(h2.at[idxu.at[k]], ub[j], sems[j]).wait()
            pltpu.make_async_copy(h2.at[idxv.at[k]], vb[j], sems[j]).wait()

        def compute(k, j):
            u, v = ub[j], vb[j]

            def grp(g, gc):
                vec = jnp.zeros((L,), jnp.float32)
                for eo in range(L):
                    e = g * L + eo
                    prods = []
                    for q in range(D // 32):
                        uu = u[e, pl.ds(q * 32, 32)]
                        vv = v[e, pl.ds(q * 32, 32)]
                        ua, ub2 = plsc.unpack(uu, format=plsc.PackFormat.INTERLEAVED)
                        va, vb2 = plsc.unpack(vv, format=plsc.PackFormat.INTERLEAVED)
                        prods.append(ua * va)
                        prods.append(ub2 * vb2)
                    a = ((prods[0] + prods[1]) + (prods[2] + prods[3])) + \
                        ((prods[4] + prods[5]) + (prods[6] + prods[7]))
                    vec = jnp.where(lane == eo, jnp.sum(a), vec)
                sres[k, pl.ds(g * L, L)] = vec
                return gc

            lax.fori_loop(0, CH // L, grp, 0)

        start_g(0, 0)

        def it(i, cc):
            start_g(2 * i + 1, 1)
            wait_g(2 * i, 0)
            compute(2 * i, 0)

            @pl.when(i < SCPT // 2 - 1)
            def _():
                start_g(2 * i + 2, 0)

            wait_g(2 * i + 1, 1)
            compute(2 * i + 1, 1)
            return cc

        lax.fori_loop(0, SCPT // 2, it, 0)
        pltpu.sync_copy(sres, sc_o.at[pl.ds(w * SCPT, SCPT)])

    return pl.kernel(body, out_type=out_type, mesh=_mesh(), scratch_types=scratch,
                     compiler_params=pltpu.CompilerParams(
                         use_tc_tiling_on_sc=False, needs_layout_passes=False))


def _t1_body(parts, deg, x, ws0, wn0, ws1, wn1, s1_o, p1_o):
    dg = deg[...]
    d = jnp.maximum(dg[0, :, 0:1] + dg[1, :, 0:1], 1.0)
    rd = 1.0 / d
    p = parts[...].astype(jnp.float32)
    wn0v = wn0[...]
    h1 = jnp.dot(x[...], ws0[...], preferred_element_type=jnp.float32)
    h1 = h1 + jnp.dot(p[0] * rd, wn0v[:DHALF], preferred_element_type=jnp.float32)
    h1 = h1 + jnp.dot(p[1] * rd, wn0v[DHALF:], preferred_element_type=jnp.float32)
    h1 = jnp.maximum(h1, 0.0)
    s1_o[...] = jnp.dot(h1, ws1[...], preferred_element_type=jnp.float32)
    wn1v = wn1[...]
    p1_o[...] = jnp.stack([
        jnp.dot(h1, wn1v[:, :DHALF], preferred_element_type=jnp.float32),
        jnp.dot(h1, wn1v[:, DHALF:], preferred_element_type=jnp.float32),
    ]).astype(jnp.bfloat16)


def _t2_body(s1, parts, deg, h2_o):
    dg = deg[...]
    d = jnp.maximum(dg[0, :, 0:1] + dg[1, :, 0:1], 1.0)
    rd = 1.0 / d
    p = parts[...].astype(jnp.float32)
    h2_o[...] = (s1[...] + jnp.concatenate([p[0], p[1]], axis=1) * rd
                 ).astype(jnp.bfloat16)


_t1_call = pl.pallas_call(
    _t1_body,
    grid=(NP // BLK,),
    in_specs=[
        pl.BlockSpec((NC, BLK, DHALF), lambda i: (0, i, 0)),
        pl.BlockSpec((NC, BLK, L), lambda i: (0, i, 0)),
        pl.BlockSpec((BLK, D), lambda i: (i, 0)),
        pl.BlockSpec((D, DH), lambda i: (0, 0)),
        pl.BlockSpec((D, DH), lambda i: (0, 0)),
        pl.BlockSpec((DH, D), lambda i: (0, 0)),
        pl.BlockSpec((DH, D), lambda i: (0, 0)),
    ],
    out_specs=[
        pl.BlockSpec((BLK, D), lambda i: (i, 0)),
        pl.BlockSpec((NC, BLK, DHALF), lambda i: (0, i, 0)),
    ],
    out_shape=[
        jax.ShapeDtypeStruct((NP, D), jnp.float32),
        jax.ShapeDtypeStruct((NC, NP, DHALF), jnp.bfloat16),
    ],
)

_t2_call = pl.pallas_call(
    _t2_body,
    grid=(NP // BLK,),
    in_specs=[
        pl.BlockSpec((BLK, D), lambda i: (i, 0)),
        pl.BlockSpec((NC, BLK, DHALF), lambda i: (0, i, 0)),
        pl.BlockSpec((NC, BLK, L), lambda i: (0, i, 0)),
    ],
    out_specs=pl.BlockSpec((BLK, D), lambda i: (i, 0)),
    out_shape=jax.ShapeDtypeStruct((NP, D), jnp.bfloat16),
)


def kernel(x, edge_index, pos_u, pos_v, neg_u, neg_v,
           W_self0, W_neigh0, W_self1, W_neigh1):
    src = edge_index[0].astype(jnp.int32)
    dst = edge_index[1].astype(jnp.int32)
    pad_e = E_PAD - E
    src2 = jnp.concatenate([src, jnp.zeros((pad_e,), jnp.int32)]).reshape(N_CHUNKS // KROW, KROW * CH)
    pad_rows = N + (jnp.arange(pad_e, dtype=jnp.int32) % (NP - N))
    dst2 = jnp.concatenate([dst, pad_rows]).reshape(N_CHUNKS // KROW, KROW * CH)
    z64 = jnp.zeros((NP, DHALF), jnp.bfloat16)
    z16 = jnp.zeros((NP, L), jnp.float32)
    ones_h = jnp.ones((KROW * CH, L), jnp.float32)
    xp = jnp.concatenate([x, jnp.zeros((NP - N, D), jnp.float32)])
    xp2 = jnp.stack([xp[:, :DHALF], xp[:, DHALF:]]).astype(jnp.bfloat16)

    seg1 = _make_segsum(with_deg=True)
    parts1, deg = seg1(xp2, src2, dst2, z64, z16, ones_h)
    s1, p1_2 = _t1_call(parts1, deg, xp, W_self0, W_neigh0, W_self1, W_neigh1)
    seg2 = _make_segsum(with_deg=False)
    (parts2,) = seg2(p1_2, src2, dst2, z64, z16, ones_h)
    h2 = _t2_call(s1, parts2, deg)

    n_pos = pos_u.shape[0]
    n_neg = neg_u.shape[0]
    pad_s = S_PAD - n_pos - n_neg
    pad_idx = jnp.arange(pad_s, dtype=jnp.int32) % N
    ui = jnp.concatenate([pos_u.astype(jnp.int32), neg_u.astype(jnp.int32),
                          pad_idx]).reshape(S_PAD // CH, CH)
    vi = jnp.concatenate([pos_v.astype(jnp.int32), neg_v.astype(jnp.int32),
                          pad_idx]).reshape(S_PAD // CH, CH)
    (scores2,) = _score_kernel()(h2, ui, vi)
    scores = scores2.reshape(-1)
    return scores[:n_pos], scores[n_pos:n_pos + n_neg]
